# baseline (speedup 1.0000x reference)
import functools

import jax
import jax.numpy as jnp
from jax import lax
from jax.experimental import pallas as pl
from jax.experimental.pallas import tpu as pltpu
from jax.experimental.pallas import tpu_sc as plsc

HIDDEN = 64
BATCH = 16384
SEQ = 50

NC = 2
NS = 16
NW = NC * NS
BPW = BATCH // NW
CB = 8
ROWS = CB * SEQ
NGRP = ROWS // 16
NCHUNK = BPW // CB
OUTW = 3 * HIDDEN
OCH = CB * OUTW
SUP = 3200
CPS = SUP // ROWS

_mesh = plsc.VectorSubcoreMesh(core_axis_name="c", subcore_axis_name="s")


@functools.partial(
    pl.kernel,
    mesh=_mesh,
    out_type=jax.ShapeDtypeStruct((BATCH * OUTW,), jnp.float32),
    scratch_types=[
        pltpu.VMEM((SUP,), jnp.int32),
        pltpu.VMEM((2 * ROWS,), jnp.int32),
        pltpu.VMEM((ROWS,), jnp.int32),
        pltpu.VMEM((2, ROWS, 128), jnp.float32),
        pltpu.VMEM((2 * OCH,), jnp.float32),
        pltpu.VMEM((OUTW,), jnp.float32),
        pltpu.SemaphoreType.DMA,
        pltpu.SemaphoreType.DMA,
        pltpu.SemaphoreType.DMA,
    ],
)
def _sc_encoder(idx_hbm, table_hbm, att_hbm, out_hbm,
                idx_sv, p64_v, ob_v, rows_v, out_v, att_v,
                sem0, sem1, osem):
    wid = lax.axis_index("c") * NS + lax.axis_index("s")
    pltpu.sync_copy(att_hbm, att_v)
    atts = [att_v[pl.ds(k * 16, 16)] for k in range(12)]
    lanes = lax.iota(jnp.int32, 16)
    perms = [(lanes ^ s)[:, None] for s in (8, 4, 2, 1)]
    _dnums = lax.GatherDimensionNumbers(
        offset_dims=(), collapsed_slice_dims=(0,), start_index_map=(0,))

    def lane_sum(v):
        for p in perms:
            v = v + lax.gather(v, p, _dnums, slice_sizes=(1,),
                               mode=lax.GatherScatterMode.PROMISE_IN_BOUNDS)
        return v

    for k in range(NGRP):
        rvec = lanes + (k * 16)
        ob_v[pl.ds(k * 16, 16)] = ((rvec * 1311) >> 16) * OUTW

    sems = (sem0, sem1)
    zeros16i = jnp.zeros((16,), jnp.int32)
    zeros16 = jnp.zeros((16,), jnp.float32)

    def fire(c, s):
        @pl.when(c % CPS == 0)
        def _():
            base = pl.multiple_of(
                wid * (BPW * SEQ) + (c // CPS) * SUP, SUP)
            pltpu.sync_copy(idx_hbm.at[pl.ds(base, SUP)], idx_sv)

        off = (c % CPS) * ROWS
        for k in range(NGRP):
            v = idx_sv[pl.ds(off + k * 16, 16)]
            p64_v[pl.ds(s * ROWS + k * 16, 16)] = (v & 1) << 6
            pltpu.make_async_copy(
                table_hbm.at[v >> 1],
                rows_v.at[s].at[pl.ds(k * 16, 16)],
                sems[s],
            ).start()

    def drain(s):
        for k in range(NGRP):
            pltpu.make_async_copy(
                table_hbm.at[zeros16i],
                rows_v.at[s].at[pl.ds(k * 16, 16)],
                sems[s],
            ).wait()

    def write_out(c, s):
        out_base = pl.multiple_of((wid * BPW + c * CB) * OUTW, OCH)
        pltpu.make_async_copy(
            out_v.at[pl.ds(s * OCH, OCH)],
            out_hbm.at[pl.ds(out_base, OCH)], osem).start()

    def drain_out(s):
        pltpu.make_async_copy(
            out_v.at[pl.ds(s * OCH, OCH)],
            out_hbm.at[pl.ds(0, OCH)], osem).wait()

    def compute(s):
        for k in range(OCH // 16):
            out_v[pl.ds(s * OCH + k * 16, 16)] = zeros16

        def g_body(k, carry):
            base = k * 16
            pvec = p64_v[pl.ds(s * ROWS + base, 16)]
            ovec = ob_v[pl.ds(base, 16)]
            for u in range(16):
                r = base + u
                p64 = pvec[u]
                ob = ovec[u] + s * OCH
                e = [rows_v[s, r, pl.ds(p64 + j * 16, 16)] for j in range(4)]
                for i in range(3):
                    m = (e[0] * atts[4 * i] + e[1] * atts[4 * i + 1]
                         + e[2] * atts[4 * i + 2] + e[3] * atts[4 * i + 3])
                    alpha = lane_sum(m)
                    for j in range(4):
                        plsc.addupdate(
                            out_v.at[pl.ds(ob + i * 64 + j * 16, 16)],
                            alpha * e[j])
            return carry

        lax.fori_loop(0, NGRP, g_body, 0)

    fire(0, 0)

    def pair_body(c2, carry):
        for par in (0, 1):
            c = c2 * 2 + par

            @pl.when(c + 1 < NCHUNK)
            def _():
                fire(c + 1, 1 - par)

            drain(par)

            @pl.when(c >= 2)
            def _():
                drain_out(par)

            compute(par)
            write_out(c, par)
        return carry

    lax.fori_loop(0, NCHUNK // 2, pair_body, 0)
    drain_out(0)
    drain_out(1)


def kernel(inputs, batch_size, embedding, attention_1, attention_2, attention_3):
    idx = inputs.reshape(-1).astype(jnp.int32)
    table = embedding.reshape(500000, 128)
    att = jnp.concatenate(
        [attention_1.reshape(-1), attention_2.reshape(-1),
         attention_3.reshape(-1)], axis=0)
    out = _sc_encoder(idx, table, att)
    return out.reshape(BATCH, 3, HIDDEN)


# speedup vs baseline: 1.2707x; 1.2707x over previous
import functools

import jax
import jax.numpy as jnp
from jax import lax
from jax.experimental import pallas as pl
from jax.experimental.pallas import tpu as pltpu
from jax.experimental.pallas import tpu_sc as plsc

HIDDEN = 64
BATCH = 16384
SEQ = 50

NC = 2
NS = 16
NW = NC * NS
BPW = BATCH // NW
CB = 8
ROWS = CB * SEQ
NGRP = ROWS // 16
NCHUNK = BPW // CB
OUTW = 3 * HIDDEN
OCH = CB * OUTW
SUP = 3200
CPS = SUP // ROWS
PADB = 64

_mesh = plsc.VectorSubcoreMesh(core_axis_name="c", subcore_axis_name="s")


@functools.partial(
    pl.kernel,
    mesh=_mesh,
    out_type=jax.ShapeDtypeStruct((BATCH * OUTW,), jnp.float32),
    scratch_types=[
        pltpu.VMEM((SUP,), jnp.int32),
        pltpu.VMEM((2 * ROWS + 16,), jnp.int32),
        pltpu.VMEM((2, ROWS, 128), jnp.float32),
        pltpu.VMEM((2 * OCH,), jnp.float32),
        pltpu.VMEM((OUTW,), jnp.float32),
        pltpu.SemaphoreType.DMA,
        pltpu.SemaphoreType.DMA,
        pltpu.SemaphoreType.DMA,
    ],
)
def _sc_encoder(idx_hbm, table_hbm, att_hbm, out_hbm,
                idx_sv, p64_v, rows_v, out_v, att_v,
                sem0, sem1, osem):
    wid = lax.axis_index("c") * NS + lax.axis_index("s")
    pltpu.sync_copy(att_hbm, att_v)
    atts = [att_v[pl.ds(k * 16, 16)] for k in range(12)]
    lanes = lax.iota(jnp.int32, 16)
    perms = [(lanes ^ s)[:, None] for s in (8, 4, 2, 1)]
    _dnums = lax.GatherDimensionNumbers(
        offset_dims=(), collapsed_slice_dims=(0,), start_index_map=(0,))

    def lane_sum(v):
        for p in perms:
            v = v + lax.gather(v, p, _dnums, slice_sizes=(1,),
                               mode=lax.GatherScatterMode.PROMISE_IN_BOUNDS)
        return v

    sems = (sem0, sem1)
    zeros16i = jnp.zeros((16,), jnp.int32)

    def fire(c, s):
        @pl.when(c % CPS == 0)
        def _():
            base = pl.multiple_of(
                wid * (BPW * SEQ) + (c // CPS) * SUP, SUP)
            pltpu.sync_copy(idx_hbm.at[pl.ds(base, SUP)], idx_sv)

        off = (c % CPS) * ROWS
        for k in range(NGRP):
            v = idx_sv[pl.ds(off + k * 16, 16)]
            p64_v[pl.ds(s * ROWS + k * 16, 16)] = (v & 1) << 6
            pltpu.make_async_copy(
                table_hbm.at[v >> 1],
                rows_v.at[s].at[pl.ds(k * 16, 16)],
                sems[s],
            ).start()

    def drain(s):
        for k in range(NGRP):
            pltpu.make_async_copy(
                table_hbm.at[zeros16i],
                rows_v.at[s].at[pl.ds(k * 16, 16)],
                sems[s],
            ).wait()

    def write_out(c, s):
        out_base = pl.multiple_of((wid * BPW + c * CB) * OUTW, OCH)
        pltpu.make_async_copy(
            out_v.at[pl.ds(s * OCH, OCH)],
            out_hbm.at[pl.ds(out_base, OCH)], osem).start()

    def drain_out(s):
        pltpu.make_async_copy(
            out_v.at[pl.ds(s * OCH, OCH)],
            out_hbm.at[pl.ds(0, OCH)], osem).wait()

    def compute(s):
        def b_body(b, carry):
            rb = b * SEQ
            pvs = [p64_v[pl.ds(s * ROWS + rb + q * 16, 16)]
                   for q in range(4)]
            acc = [jnp.zeros((16,), jnp.float32) for _ in range(12)]
            for l in range(SEQ):
                p64 = pvs[l // 16][l % 16]
                e = [rows_v[s, rb + l, pl.ds(p64 + j * 16, 16)]
                     for j in range(4)]
                for i in range(3):
                    m = (e[0] * atts[4 * i] + e[1] * atts[4 * i + 1]
                         + e[2] * atts[4 * i + 2] + e[3] * atts[4 * i + 3])
                    alpha = lane_sum(m)
                    for j in range(4):
                        acc[4 * i + j] = acc[4 * i + j] + alpha * e[j]
            ob = pl.multiple_of(b * OUTW, OUTW) + s * OCH
            for k in range(12):
                out_v[pl.ds(ob + k * 16, 16)] = acc[k]
            return carry

        lax.fori_loop(0, CB, b_body, 0)

    fire(0, 0)

    def pair_body(c2, carry):
        for par in (0, 1):
            c = c2 * 2 + par

            @pl.when(c + 1 < NCHUNK)
            def _():
                fire(c + 1, 1 - par)

            drain(par)

            @pl.when(c >= 2)
            def _():
                drain_out(par)

            compute(par)
            write_out(c, par)
        return carry

    lax.fori_loop(0, NCHUNK // 2, pair_body, 0)
    drain_out(0)
    drain_out(1)


def kernel(inputs, batch_size, embedding, attention_1, attention_2, attention_3):
    idx = inputs.reshape(-1).astype(jnp.int32)
    table = embedding.reshape(500000, 128)
    att = jnp.concatenate(
        [attention_1.reshape(-1), attention_2.reshape(-1),
         attention_3.reshape(-1)], axis=0)
    out = _sc_encoder(idx, table, att)
    return out.reshape(BATCH, 3, HIDDEN)


# speedup vs baseline: 1.3733x; 1.0808x over previous
import functools

import jax
import jax.numpy as jnp
from jax import lax
from jax.experimental import pallas as pl
from jax.experimental.pallas import tpu as pltpu
from jax.experimental.pallas import tpu_sc as plsc

HIDDEN = 64
BATCH = 16384
SEQ = 50

NC = 2
NS = 16
NW = NC * NS
BPW = BATCH // NW
CB = 8
ROWS = CB * SEQ
NGRP = ROWS // 16
NCHUNK = BPW // CB
OUTW = 3 * HIDDEN
OCH = CB * OUTW
SUP = 3200
CPS = SUP // ROWS
PADB = 64

_mesh = plsc.VectorSubcoreMesh(core_axis_name="c", subcore_axis_name="s")


@functools.partial(
    pl.kernel,
    mesh=_mesh,
    out_type=jax.ShapeDtypeStruct((BATCH * OUTW,), jnp.float32),
    scratch_types=[
        pltpu.VMEM((SUP,), jnp.int32),
        pltpu.VMEM((2 * ROWS, 128), jnp.float32),
        pltpu.VMEM((2 * OCH,), jnp.float32),
        pltpu.VMEM((OUTW,), jnp.float32),
        pltpu.SemaphoreType.DMA,
        pltpu.SemaphoreType.DMA,
        pltpu.SemaphoreType.DMA,
    ],
)
def _sc_encoder(idx_hbm, table_hbm, att_hbm, out_hbm,
                idx_sv, rows_v, out_v, att_v,
                sem0, sem1, osem):
    wid = lax.axis_index("c") * NS + lax.axis_index("s")
    pltpu.sync_copy(att_hbm, att_v)
    atts = [att_v[pl.ds(k * 16, 16)] for k in range(12)]
    lanes = lax.iota(jnp.int32, 16)
    perms = [(lanes ^ s)[:, None] for s in (8, 4, 2, 1)]
    _dnums = lax.GatherDimensionNumbers(
        offset_dims=(), collapsed_slice_dims=(0,), start_index_map=(0,))

    def lane_sum(v):
        for p in perms:
            v = v + lax.gather(v, p, _dnums, slice_sizes=(1,),
                               mode=lax.GatherScatterMode.PROMISE_IN_BOUNDS)
        return v

    sems = (sem0, sem1)
    zeros16i = jnp.zeros((16,), jnp.int32)

    def fire(c, s):
        @pl.when(c % CPS == 0)
        def _():
            base = pl.multiple_of(
                wid * (BPW * SEQ) + (c // CPS) * SUP, SUP)
            pltpu.sync_copy(idx_hbm.at[pl.ds(base, SUP)], idx_sv)

        off = (c % CPS) * ROWS
        for k in range(NGRP):
            v = idx_sv[pl.ds(off + k * 16, 16)]
            pltpu.make_async_copy(
                table_hbm.at[v],
                rows_v.at[pl.ds(s * ROWS + k * 16, 16)],
                sems[s],
            ).start()

    def drain(s):
        for k in range(NGRP):
            pltpu.make_async_copy(
                table_hbm.at[zeros16i],
                rows_v.at[pl.ds(s * ROWS + k * 16, 16)],
                sems[s],
            ).wait()

    def write_out(c, s):
        out_base = pl.multiple_of((wid * BPW + c * CB) * OUTW, OCH)
        pltpu.make_async_copy(
            out_v.at[pl.ds(s * OCH, OCH)],
            out_hbm.at[pl.ds(out_base, OCH)], osem).start()

    def drain_out(s):
        pltpu.make_async_copy(
            out_v.at[pl.ds(s * OCH, OCH)],
            out_hbm.at[pl.ds(0, OCH)], osem).wait()

    def compute(s):
        def b_body(b, carry):
            rb = s * ROWS + b * SEQ
            acc = [jnp.zeros((16,), jnp.float32) for _ in range(12)]
            for l in range(SEQ):
                e = [rows_v[rb + l, pl.ds(j * 16, 16)]
                     for j in range(4)]
                for i in range(3):
                    m = (e[0] * atts[4 * i] + e[1] * atts[4 * i + 1]
                         + e[2] * atts[4 * i + 2] + e[3] * atts[4 * i + 3])
                    alpha = lane_sum(m)
                    for j in range(4):
                        acc[4 * i + j] = acc[4 * i + j] + alpha * e[j]
            ob = pl.multiple_of(b * OUTW, OUTW) + s * OCH
            for k in range(12):
                out_v[pl.ds(ob + k * 16, 16)] = acc[k]
            return carry

        lax.fori_loop(0, CB, b_body, 0)

    fire(0, 0)

    def pair_body(c2, carry):
        for par in (0, 1):
            c = c2 * 2 + par

            @pl.when(c + 1 < NCHUNK)
            def _():
                fire(c + 1, 1 - par)

            drain(par)

            @pl.when(c >= 2)
            def _():
                drain_out(par)

            compute(par)
            write_out(c, par)
        return carry

    lax.fori_loop(0, NCHUNK // 2, pair_body, 0)
    drain_out(0)
    drain_out(1)


def kernel(inputs, batch_size, embedding, attention_1, attention_2, attention_3):
    idx = inputs.reshape(-1).astype(jnp.int32)
    table = jnp.pad(embedding, ((0, 0), (0, 64)))
    att = jnp.concatenate(
        [attention_1.reshape(-1), attention_2.reshape(-1),
         attention_3.reshape(-1)], axis=0)
    out = _sc_encoder(idx, table, att)
    return out.reshape(BATCH, 3, HIDDEN)
